# baseline (speedup 1.0000x reference)
import functools

import jax
import jax.numpy as jnp
from jax import lax
from jax.experimental import pallas as pl
from jax.experimental.pallas import tpu as pltpu
from jax.experimental.pallas import tpu_sc as plsc

_INFO = plsc.get_sparse_core_info()
_NC = _INFO.num_cores
_NS = _INFO.num_subcores
_NW = _NC * _NS

_CHUNK = 640
_VBLK = 128


def _format_table(table_t, tail_flat, vocab, d):
    nblk_full = vocab // _VBLK
    per_w = nblk_full // _NW
    n_extra = nblk_full - per_w * _NW
    tail_rows = vocab - nblk_full * _VBLK
    blk_elems = _VBLK * d

    mesh = plsc.VectorSubcoreMesh(core_axis_name="c", subcore_axis_name="s")

    @functools.partial(
        pl.kernel,
        out_type=jax.ShapeDtypeStruct((vocab * d,), jnp.float32),
        mesh=mesh,
        scratch_types=[
            pltpu.VMEM((d, _VBLK), jnp.float32),
            pltpu.VMEM((d, _VBLK), jnp.float32),
            pltpu.VMEM((d, _VBLK), jnp.float32),
            pltpu.VMEM((d, _VBLK), jnp.float32),
            pltpu.VMEM((blk_elems,), jnp.float32),
            pltpu.VMEM((blk_elems,), jnp.float32),
            pltpu.VMEM((blk_elems,), jnp.float32),
            pltpu.VMEM((blk_elems,), jnp.float32),
            pltpu.VMEM((tail_rows * d,), jnp.float32),
            pltpu.SemaphoreType.DMA,
            pltpu.SemaphoreType.DMA,
            pltpu.SemaphoreType.DMA,
            pltpu.SemaphoreType.DMA,
        ],
        compiler_params=pltpu.CompilerParams(use_tc_tiling_on_sc=True,
                                             needs_layout_passes=False),
    )
    def k1(tab_hbm, tail_hbm, out_hbm, ibA0, ibA1, ibB0, ibB1,
           obA0, obA1, obB0, obB1, tb, lsemA, lsemB, osemA, osemB):
        wid = lax.axis_index("s") * _NC + lax.axis_index("c")
        blk0 = wid * per_w
        iota = lax.iota(jnp.int32, 16)

        def transpose_block(ib, ob):
            for g in range(d // 16):
                c0 = iota + 16 * g

                @plsc.parallel_loop(0, _VBLK - 15, unroll=4)
                def _(m):
                    vec = plsc.load_gather(ib, [c0, iota + m])
                    plsc.store_scatter(
                        ob, [iota * (d + 1) + (m * d + 16 * g)], vec)

                for m in list(range(-15, 0)) + list(range(_VBLK - 15, _VBLK)):
                    rr = iota + m
                    lanes = (rr >= 0) & (rr < _VBLK)
                    r_idx = jnp.clip(rr, 0, _VBLK - 1)
                    vec = plsc.load_gather(ib, [c0, r_idx], mask=lanes)
                    s_idx = jnp.clip(iota * (d + 1) + (m * d + 16 * g),
                                     0, blk_elems - 1)
                    plsc.store_scatter(ob, [s_idx], vec, mask=lanes)

        def load_block(blk, ib, sem):
            return pltpu.async_copy(
                tab_hbm.at[:, pl.ds(blk * _VBLK, _VBLK)], ib, sem)

        def store_block(blk, ob, sem):
            return pltpu.async_copy(
                ob, out_hbm.at[pl.ds(blk * blk_elems, blk_elems)], sem)

        def drain_load(ib, sem):
            pltpu.make_async_copy(
                tab_hbm.at[:, pl.ds(0, _VBLK)], ib, sem).wait()

        def drain_store(ob, sem):
            pltpu.make_async_copy(out_hbm.at[pl.ds(0, blk_elems)], ob,
                                  sem).wait()

        load_block(blk0, ibA0, lsemA)
        load_block(blk0 + 1, ibA1, lsemA)

        def body(j, carry):
            b = blk0 + 4 * j
            load_block(b + 2, ibB0, lsemB)
            load_block(b + 3, ibB1, lsemB)
            drain_load(ibA0, lsemA)
            drain_load(ibA1, lsemA)

            @pl.when(j > 0)
            def _():
                drain_store(obA0, osemA)
                drain_store(obA1, osemA)

            transpose_block(ibA0, obA0)
            store_block(b, obA0, osemA)
            transpose_block(ibA1, obA1)
            store_block(b + 1, obA1, osemA)

            load_block(b + 4, ibA0, lsemA)
            load_block(b + 5, ibA1, lsemA)
            drain_load(ibB0, lsemB)
            drain_load(ibB1, lsemB)

            @pl.when(j > 0)
            def _():
                drain_store(obB0, osemB)
                drain_store(obB1, osemB)

            transpose_block(ibB0, obB0)
            store_block(b + 2, obB0, osemB)
            transpose_block(ibB1, obB1)
            store_block(b + 3, obB1, osemB)
            return carry

        lax.fori_loop(0, per_w // 4, body, 0)
        drain_load(ibA0, lsemA)
        drain_load(ibA1, lsemA)
        drain_store(obA0, osemA)
        drain_store(obA1, osemA)
        drain_store(obB0, osemB)
        drain_store(obB1, osemB)

        @pl.when(wid < n_extra)
        def _():
            blk = nblk_full - n_extra + wid
            load_block(blk, ibA0, lsemA).wait()
            transpose_block(ibA0, obA0)
            store_block(blk, obA0, osemA)
            drain_store(obA0, osemA)

        @pl.when(wid == n_extra)
        def _():
            pltpu.sync_copy(tail_hbm, tb)
            pltpu.sync_copy(
                tb, out_hbm.at[pl.ds(nblk_full * blk_elems, tail_rows * d)])

    return k1(table_t, tail_flat)


def _gather_flat(idx_flat, table_lin, n_rows, d):
    b_per_w = n_rows // _NW
    nchunks = b_per_w // _CHUNK
    assert nchunks * _CHUNK == b_per_w

    mesh = plsc.VectorSubcoreMesh(core_axis_name="c", subcore_axis_name="s")

    @functools.partial(
        pl.kernel,
        out_type=jax.ShapeDtypeStruct((n_rows, d), jnp.float32),
        mesh=mesh,
        scratch_types=[
            pltpu.VMEM((b_per_w,), jnp.int32),
            pltpu.VMEM((_CHUNK, d), jnp.float32),
            pltpu.VMEM((_CHUNK, d), jnp.float32),
            pltpu.SemaphoreType.DMA,
            pltpu.SemaphoreType.DMA,
            pltpu.SemaphoreType.DMA,
            pltpu.SemaphoreType.DMA,
        ],
        compiler_params=pltpu.CompilerParams(use_tc_tiling_on_sc=False),
    )
    def k2(idx_hbm, table_hbm, out_hbm, idx_v, rows0, rows1, g0, g1, o0, o1):
        wid = lax.axis_index("s") * _NC + lax.axis_index("c")
        base = wid * b_per_w
        pltpu.sync_copy(idx_hbm.at[pl.ds(base, b_per_w)], idx_v)

        rows = (rows0, rows1)
        gsem = (g0, g1)
        osem = (o0, o1)

        def start_gather(j):
            b = j % 2
            return pltpu.async_copy(
                table_hbm.at[idx_v.at[pl.ds(j * _CHUNK, _CHUNK)]],
                rows[b], gsem[b])

        gh = [None, None]
        oh = [None, None]
        gh[0] = start_gather(0)
        for j in range(nchunks):
            b = j % 2
            nb = (j + 1) % 2
            if j + 1 < nchunks:
                if oh[nb] is not None:
                    oh[nb].wait()
                gh[nb] = start_gather(j + 1)
            gh[b].wait()
            oh[b] = pltpu.async_copy(
                rows[b], out_hbm.at[pl.ds(base + j * _CHUNK, _CHUNK)],
                osem[b])
        for h in oh:
            if h is not None:
                h.wait()

    return k2(idx_flat, table_lin)


def kernel(input_ids, emb_table):
    bsz, seq = input_ids.shape
    vocab, d = emb_table.shape
    n_rows = bsz * seq
    idx_flat = input_ids.reshape(n_rows).astype(jnp.int32)
    nblk_full = vocab // _VBLK
    tail_flat = emb_table[nblk_full * _VBLK:, :].reshape(-1)
    table_lin = _format_table(emb_table.T, tail_flat, vocab, d)
    out = _gather_flat(idx_flat, table_lin.reshape(vocab, d), n_rows, d)
    return out.reshape(bsz, seq, d)


# speedup vs baseline: 1.0168x; 1.0168x over previous
import functools

import jax
import jax.numpy as jnp
from jax import lax
from jax.experimental import pallas as pl
from jax.experimental.pallas import tpu as pltpu
from jax.experimental.pallas import tpu_sc as plsc

_INFO = plsc.get_sparse_core_info()
_NC = _INFO.num_cores
_NS = _INFO.num_subcores
_NW = _NC * _NS

_CHUNK = 640
_VBLK = 128


def _format_table(table_t, tail_flat, vocab, d):
    nblk_full = vocab // _VBLK
    per_w = nblk_full // _NW
    n_extra = nblk_full - per_w * _NW
    tail_rows = vocab - nblk_full * _VBLK
    blk_elems = _VBLK * d

    mesh = plsc.VectorSubcoreMesh(core_axis_name="c", subcore_axis_name="s")

    @functools.partial(
        pl.kernel,
        out_type=jax.ShapeDtypeStruct((vocab * d,), jnp.float32),
        mesh=mesh,
        scratch_types=[
            pltpu.VMEM((d, _VBLK), jnp.float32),
            pltpu.VMEM((d, _VBLK), jnp.float32),
            pltpu.VMEM((d, _VBLK), jnp.float32),
            pltpu.VMEM((d, _VBLK), jnp.float32),
            pltpu.VMEM((blk_elems,), jnp.float32),
            pltpu.VMEM((blk_elems,), jnp.float32),
            pltpu.VMEM((blk_elems,), jnp.float32),
            pltpu.VMEM((blk_elems,), jnp.float32),
            pltpu.VMEM((tail_rows * d,), jnp.float32),
            pltpu.SemaphoreType.DMA,
            pltpu.SemaphoreType.DMA,
            pltpu.SemaphoreType.DMA,
            pltpu.SemaphoreType.DMA,
        ],
        compiler_params=pltpu.CompilerParams(use_tc_tiling_on_sc=True,
                                             needs_layout_passes=False),
    )
    def k1(tab_hbm, tail_hbm, out_hbm, ibA0, ibA1, ibB0, ibB1,
           obA0, obA1, obB0, obB1, tb, lsemA, lsemB, osemA, osemB):
        wid = lax.axis_index("s") * _NC + lax.axis_index("c")
        blk0 = wid * per_w
        iota = lax.iota(jnp.int32, 16)
        c0s = [iota + 16 * g for g in range(d // 16)]

        def transpose_block(ib, ob):
            @plsc.parallel_loop(0, _VBLK - 15, unroll=8)
            def _(m):
                rr = iota + m
                for g in range(d // 16):
                    vec = plsc.load_gather(ib, [c0s[g], rr])
                    plsc.store_scatter(
                        ob, [iota * (d + 1) + (m * d + 16 * g)], vec)

            for m in list(range(-15, 0)) + list(range(_VBLK - 15, _VBLK)):
                rr = iota + m
                lanes = (rr >= 0) & (rr < _VBLK)
                r_idx = jnp.clip(rr, 0, _VBLK - 1)
                for g in range(d // 16):
                    vec = plsc.load_gather(ib, [c0s[g], r_idx], mask=lanes)
                    s_idx = jnp.clip(iota * (d + 1) + (m * d + 16 * g),
                                     0, blk_elems - 1)
                    plsc.store_scatter(ob, [s_idx], vec, mask=lanes)

        def load_block(blk, ib, sem):
            return pltpu.async_copy(
                tab_hbm.at[:, pl.ds(blk * _VBLK, _VBLK)], ib, sem)

        def store_block(blk, ob, sem):
            return pltpu.async_copy(
                ob, out_hbm.at[pl.ds(blk * blk_elems, blk_elems)], sem)

        def drain_load(ib, sem):
            pltpu.make_async_copy(
                tab_hbm.at[:, pl.ds(0, _VBLK)], ib, sem).wait()

        def drain_store(ob, sem):
            pltpu.make_async_copy(out_hbm.at[pl.ds(0, blk_elems)], ob,
                                  sem).wait()

        load_block(blk0, ibA0, lsemA)
        load_block(blk0 + 1, ibA1, lsemA)

        def body(j, carry):
            b = blk0 + 4 * j
            load_block(b + 2, ibB0, lsemB)
            load_block(b + 3, ibB1, lsemB)
            drain_load(ibA0, lsemA)
            drain_load(ibA1, lsemA)

            @pl.when(j > 0)
            def _():
                drain_store(obA0, osemA)
                drain_store(obA1, osemA)

            transpose_block(ibA0, obA0)
            store_block(b, obA0, osemA)
            transpose_block(ibA1, obA1)
            store_block(b + 1, obA1, osemA)

            load_block(b + 4, ibA0, lsemA)
            load_block(b + 5, ibA1, lsemA)
            drain_load(ibB0, lsemB)
            drain_load(ibB1, lsemB)

            @pl.when(j > 0)
            def _():
                drain_store(obB0, osemB)
                drain_store(obB1, osemB)

            transpose_block(ibB0, obB0)
            store_block(b + 2, obB0, osemB)
            transpose_block(ibB1, obB1)
            store_block(b + 3, obB1, osemB)
            return carry

        lax.fori_loop(0, per_w // 4, body, 0)
        drain_load(ibA0, lsemA)
        drain_load(ibA1, lsemA)
        drain_store(obA0, osemA)
        drain_store(obA1, osemA)
        drain_store(obB0, osemB)
        drain_store(obB1, osemB)

        @pl.when(wid < n_extra)
        def _():
            blk = nblk_full - n_extra + wid
            load_block(blk, ibA0, lsemA).wait()
            transpose_block(ibA0, obA0)
            store_block(blk, obA0, osemA)
            drain_store(obA0, osemA)

        @pl.when(wid == n_extra)
        def _():
            pltpu.sync_copy(tail_hbm, tb)
            pltpu.sync_copy(
                tb, out_hbm.at[pl.ds(nblk_full * blk_elems, tail_rows * d)])

    return k1(table_t, tail_flat)


def _gather_flat(idx_flat, table_lin, n_rows, d):
    b_per_w = n_rows // _NW
    nchunks = b_per_w // _CHUNK
    assert nchunks * _CHUNK == b_per_w

    mesh = plsc.VectorSubcoreMesh(core_axis_name="c", subcore_axis_name="s")

    @functools.partial(
        pl.kernel,
        out_type=jax.ShapeDtypeStruct((n_rows, d), jnp.float32),
        mesh=mesh,
        scratch_types=[
            pltpu.VMEM((b_per_w,), jnp.int32),
            pltpu.VMEM((_CHUNK, d), jnp.float32),
            pltpu.VMEM((_CHUNK, d), jnp.float32),
            pltpu.SemaphoreType.DMA,
            pltpu.SemaphoreType.DMA,
            pltpu.SemaphoreType.DMA,
            pltpu.SemaphoreType.DMA,
        ],
        compiler_params=pltpu.CompilerParams(use_tc_tiling_on_sc=False),
    )
    def k2(idx_hbm, table_hbm, out_hbm, idx_v, rows0, rows1, g0, g1, o0, o1):
        wid = lax.axis_index("s") * _NC + lax.axis_index("c")
        base = wid * b_per_w
        pltpu.sync_copy(idx_hbm.at[pl.ds(base, b_per_w)], idx_v)

        rows = (rows0, rows1)
        gsem = (g0, g1)
        osem = (o0, o1)

        def start_gather(j):
            b = j % 2
            return pltpu.async_copy(
                table_hbm.at[idx_v.at[pl.ds(j * _CHUNK, _CHUNK)]],
                rows[b], gsem[b])

        gh = [None, None]
        oh = [None, None]
        gh[0] = start_gather(0)
        for j in range(nchunks):
            b = j % 2
            nb = (j + 1) % 2
            if j + 1 < nchunks:
                if oh[nb] is not None:
                    oh[nb].wait()
                gh[nb] = start_gather(j + 1)
            gh[b].wait()
            oh[b] = pltpu.async_copy(
                rows[b], out_hbm.at[pl.ds(base + j * _CHUNK, _CHUNK)],
                osem[b])
        for h in oh:
            if h is not None:
                h.wait()

    return k2(idx_flat, table_lin)


def kernel(input_ids, emb_table):
    bsz, seq = input_ids.shape
    vocab, d = emb_table.shape
    n_rows = bsz * seq
    idx_flat = input_ids.reshape(n_rows).astype(jnp.int32)
    nblk_full = vocab // _VBLK
    tail_flat = emb_table[nblk_full * _VBLK:, :].reshape(-1)
    table_lin = _format_table(emb_table.T, tail_flat, vocab, d)
    out = _gather_flat(idx_flat, table_lin.reshape(vocab, d), n_rows, d)
    return out.reshape(bsz, seq, d)
